# baseline (speedup 1.0000x reference)
import functools
import jax
import jax.numpy as jnp
from jax import lax
from jax.experimental import pallas as pl
from jax.experimental.pallas import tpu as pltpu
from jax.experimental.pallas import tpu_sc as plsc

_N = 10000
_E = 320000
_M = 80000
_IN = 128
_HID = 256

_F = 16
_NB = _HID // _F
_NCORE = 2
_NSUB = 16
_BATCH = 128
_JTILE = 160
_EPAD = _JTILE * _NSUB * _BATCH
_ME = 81920
_CH = 512
_ECH = _ME // _NSUB // _CH
_NV = 10112
_NP = 10016


def _mm_kernel(x_ref, w_ref, o_ref):
    o_ref[...] = jnp.dot(x_ref[...], w_ref[...],
                         preferred_element_type=jnp.float32)


def _matmul(X, W):
    BM = 1000
    return pl.pallas_call(
        _mm_kernel,
        grid=(_N // BM,),
        in_specs=[
            pl.BlockSpec((BM, _IN), lambda i: (i, 0)),
            pl.BlockSpec((_IN, _HID), lambda i: (0, 0)),
        ],
        out_specs=pl.BlockSpec((BM, _HID), lambda i: (i, 0)),
        out_shape=jax.ShapeDtypeStruct((_N, _HID), jnp.float32),
    )(X, W)


def _ep_kernel(eps_ref, xh_ref, xv_ref, o_ref):
    o = (1.0 + eps_ref[0]) * xh_ref[...] + xv_ref[...]
    ss = jnp.sum(o * o, axis=1, keepdims=True)
    rn = jnp.sqrt(ss)
    scale = jnp.where(rn > 0, 1.0 / rn, 0.0)
    o_ref[...] = o * scale


def _epilogue(eps, Xh, Xv):
    BM = 1000
    return pl.pallas_call(
        _ep_kernel,
        grid=(_N // BM,),
        in_specs=[
            pl.BlockSpec(memory_space=pltpu.SMEM),
            pl.BlockSpec((BM, _HID), lambda i: (i, 0)),
            pl.BlockSpec((BM, _HID), lambda i: (i, 0)),
        ],
        out_specs=pl.BlockSpec((BM, _HID), lambda i: (i, 0)),
        out_shape=jax.ShapeDtypeStruct((_N, _HID), jnp.float32),
    )(eps, Xh, Xv)


def _sc_body(xh, v2d, e2d, zsrc, osrc, out, invc,
             acc_e, acc_v, ivb, ieb, rows, sbuf, jbuf, zbuf, onesb):
    c = lax.axis_index("c")
    s = lax.axis_index("s")

    ebase = s * (_ME // _NSUB)
    jbase = s * _JTILE

    pltpu.sync_copy(zsrc, zbuf)
    pltpu.sync_copy(osrc, onesb)

    def zero_acc_e():
        for k in range(_ECH):
            pltpu.sync_copy(zbuf, acc_e.at[pl.ds(ebase + k * _CH, _CH)])

    zero_acc_e()
    plsc.subcore_barrier()

    def cbody(j, carry):
        pltpu.sync_copy(e2d.at[jbase + j], ieb)
        pltpu.sync_copy(onesb, acc_e.at[ieb], add=True)
        return carry
    lax.fori_loop(0, _JTILE, cbody, 0)
    plsc.subcore_barrier()

    for k in range(_ECH):
        pltpu.sync_copy(acc_e.at[pl.ds(ebase + k * _CH, _CH)], sbuf)

        def gbody(r, carry):
            sbuf[r, :] = 1.0 / jnp.maximum(sbuf[r, :], 1.0)
            return carry
        lax.fori_loop(0, _CH, gbody, 0)
        pltpu.sync_copy(sbuf, invc.at[c].at[pl.ds(ebase + k * _CH, _CH)])
    plsc.subcore_barrier()

    for bl in range(_NB // _NCORE):
        bg = c * (_NB // _NCORE) + bl
        zero_acc_e()
        pltpu.sync_copy(zbuf, acc_v.at[pl.ds(s * 632, _CH)])
        pltpu.sync_copy(zbuf.at[pl.ds(0, 120)],
                        acc_v.at[pl.ds(s * 632 + _CH, 120)])
        plsc.subcore_barrier()

        def s1(j, carry):
            pltpu.sync_copy(v2d.at[jbase + j], ivb)
            pltpu.sync_copy(e2d.at[jbase + j], ieb)
            pltpu.sync_copy(xh.at[bg].at[ivb], rows)
            pltpu.sync_copy(rows, acc_e.at[ieb], add=True)
            return carry
        lax.fori_loop(0, _JTILE, s1, 0)
        plsc.subcore_barrier()

        for k in range(_ECH):
            pltpu.sync_copy(acc_e.at[pl.ds(ebase + k * _CH, _CH)], sbuf)
            pltpu.sync_copy(invc.at[c].at[pl.ds(ebase + k * _CH, _CH)], jbuf)

            def scbody(r, carry):
                sbuf[r, :] = sbuf[r, :] * jbuf[r, :]
                return carry
            lax.fori_loop(0, _CH, scbody, 0)
            pltpu.sync_copy(sbuf, acc_e.at[pl.ds(ebase + k * _CH, _CH)])
        plsc.subcore_barrier()

        def s2(j, carry):
            pltpu.sync_copy(v2d.at[jbase + j], ivb)
            pltpu.sync_copy(e2d.at[jbase + j], ieb)
            pltpu.sync_copy(acc_e.at[ieb], rows)
            pltpu.sync_copy(rows, acc_v.at[ivb], add=True)
            return carry
        lax.fori_loop(0, _JTILE, s2, 0)
        plsc.subcore_barrier()

        @pl.when(s < _NSUB - 1)
        def _():
            pltpu.sync_copy(acc_v.at[pl.ds(s * 624, 624)],
                            out.at[bg].at[pl.ds(s * 624, 624)])

        @pl.when(s == _NSUB - 1)
        def _():
            pltpu.sync_copy(acc_v.at[pl.ds(15 * 624, 640)],
                            out.at[bg].at[pl.ds(15 * 624, 640)])
        plsc.subcore_barrier()


_sc_call = pl.kernel(
    _sc_body,
    out_type=(
        jax.ShapeDtypeStruct((_NB, _N, _F), jnp.float32),
        jax.ShapeDtypeStruct((_NCORE, _ME, _F), jnp.float32),
    ),
    mesh=plsc.VectorSubcoreMesh(core_axis_name="c", subcore_axis_name="s"),
    compiler_params=pltpu.CompilerParams(use_tc_tiling_on_sc=False),
    scratch_types=[
        pltpu.VMEM_SHARED((_ME, _F), jnp.float32),
        pltpu.VMEM_SHARED((_NV, _F), jnp.float32),
        pltpu.VMEM((_BATCH,), jnp.int32),
        pltpu.VMEM((_BATCH,), jnp.int32),
        pltpu.VMEM((_BATCH, _F), jnp.float32),
        pltpu.VMEM((_CH, _F), jnp.float32),
        pltpu.VMEM((_CH, _F), jnp.float32),
        pltpu.VMEM((_CH, _F), jnp.float32),
        pltpu.VMEM((_BATCH, _F), jnp.float32),
    ],
)


def kernel(X, vertex, edges, W, eps):
    Xh = _matmul(X, W)

    xh_pad = jnp.concatenate(
        [Xh, jnp.zeros((_NP - _N, _HID), jnp.float32)], axis=0)
    xh_b = xh_pad.reshape(_NP, _NB, _F).transpose(1, 0, 2)

    pad = _EPAD - _E
    v2d = jnp.concatenate(
        [vertex.astype(jnp.int32), jnp.full((pad,), _N, jnp.int32)]
    ).reshape(_EPAD // _BATCH, _BATCH)
    e2d = jnp.concatenate(
        [edges.astype(jnp.int32), jnp.full((pad,), _M, jnp.int32)]
    ).reshape(_EPAD // _BATCH, _BATCH)

    zsrc = jnp.zeros((_CH, _F), jnp.float32)
    osrc = jnp.ones((_BATCH, _F), jnp.float32)

    Xv_b, _unused_invc = _sc_call(xh_b, v2d, e2d, zsrc, osrc)
    Xv = Xv_b.transpose(1, 0, 2).reshape(_N, _HID)
    return _epilogue(eps, Xh, Xv)


# speedup vs baseline: 2.2893x; 2.2893x over previous
import functools
import jax
import jax.numpy as jnp
from jax import lax
from jax.experimental import pallas as pl
from jax.experimental.pallas import tpu as pltpu
from jax.experimental.pallas import tpu_sc as plsc

_N = 10000
_E = 320000
_M = 80000
_IN = 128
_HID = 256

_F = 16
_NB = _HID // _F
_NCORE = 2
_NSUB = 16
_BATCH = 1024
_JTILE = 20
_EPAD = _JTILE * _NSUB * _BATCH
_ME = 81920
_CH = 512
_ECH = _ME // _NSUB // _CH
_NV = 10112
_NP = 10016


def _mm_kernel(x_ref, w_ref, o_ref):
    o_ref[...] = jnp.dot(x_ref[...], w_ref[...],
                         preferred_element_type=jnp.float32)


def _matmul(X, W):
    BM = 1000
    return pl.pallas_call(
        _mm_kernel,
        grid=(_N // BM,),
        in_specs=[
            pl.BlockSpec((BM, _IN), lambda i: (i, 0)),
            pl.BlockSpec((_IN, _HID), lambda i: (0, 0)),
        ],
        out_specs=pl.BlockSpec((BM, _HID), lambda i: (i, 0)),
        out_shape=jax.ShapeDtypeStruct((_N, _HID), jnp.float32),
    )(X, W)


def _ep_kernel(eps_ref, xh_ref, xv_ref, o_ref):
    o = (1.0 + eps_ref[0]) * xh_ref[...] + xv_ref[...]
    ss = jnp.sum(o * o, axis=1, keepdims=True)
    rn = jnp.sqrt(ss)
    scale = jnp.where(rn > 0, 1.0 / rn, 0.0)
    o_ref[...] = o * scale


def _epilogue(eps, Xh, Xv):
    BM = 1000
    return pl.pallas_call(
        _ep_kernel,
        grid=(_N // BM,),
        in_specs=[
            pl.BlockSpec(memory_space=pltpu.SMEM),
            pl.BlockSpec((BM, _HID), lambda i: (i, 0)),
            pl.BlockSpec((BM, _HID), lambda i: (i, 0)),
        ],
        out_specs=pl.BlockSpec((BM, _HID), lambda i: (i, 0)),
        out_shape=jax.ShapeDtypeStruct((_N, _HID), jnp.float32),
    )(eps, Xh, Xv)


def _sc_body(xh, v2d, e2d, zsrc, osrc, out, invc,
             acc_e, acc_v, ivb, ieb, rows, sbuf, jbuf):
    c = lax.axis_index("c")
    s = lax.axis_index("s")

    ebase = s * (_ME // _NSUB)
    jbase = s * _JTILE

    def zero_acc_e():
        pltpu.sync_copy(zsrc, sbuf)
        for k in range(_ECH):
            pltpu.sync_copy(sbuf, acc_e.at[pl.ds(ebase + k * _CH, _CH)])

    zero_acc_e()
    plsc.subcore_barrier()

    pltpu.sync_copy(osrc, rows)

    def cbody(j, carry):
        pltpu.sync_copy(e2d.at[jbase + j], ieb)
        pltpu.sync_copy(rows, acc_e.at[ieb], add=True)
        return carry
    lax.fori_loop(0, _JTILE, cbody, 0)
    plsc.subcore_barrier()

    for k in range(_ECH):
        pltpu.sync_copy(acc_e.at[pl.ds(ebase + k * _CH, _CH)], sbuf)

        def gbody(r, carry):
            sbuf[r, :] = 1.0 / jnp.maximum(sbuf[r, :], 1.0)
            return carry
        lax.fori_loop(0, _CH, gbody, 0)
        pltpu.sync_copy(sbuf, invc.at[c].at[pl.ds(ebase + k * _CH, _CH)])
    plsc.subcore_barrier()

    for bl in range(_NB // _NCORE):
        bg = c * (_NB // _NCORE) + bl
        zero_acc_e()
        pltpu.sync_copy(sbuf, acc_v.at[pl.ds(s * 632, _CH)])
        pltpu.sync_copy(sbuf.at[pl.ds(0, 120)],
                        acc_v.at[pl.ds(s * 632 + _CH, 120)])
        plsc.subcore_barrier()

        def s1(j, carry):
            pltpu.sync_copy(v2d.at[jbase + j], ivb)
            pltpu.sync_copy(e2d.at[jbase + j], ieb)
            pltpu.sync_copy(xh.at[bg].at[ivb], rows)
            pltpu.sync_copy(rows, acc_e.at[ieb], add=True)
            return carry
        lax.fori_loop(0, _JTILE, s1, 0)
        plsc.subcore_barrier()

        for k in range(_ECH):
            pltpu.sync_copy(acc_e.at[pl.ds(ebase + k * _CH, _CH)], sbuf)
            pltpu.sync_copy(invc.at[c].at[pl.ds(ebase + k * _CH, _CH)], jbuf)

            def scbody(r, carry):
                sbuf[r, :] = sbuf[r, :] * jbuf[r, :]
                return carry
            lax.fori_loop(0, _CH, scbody, 0)
            pltpu.sync_copy(sbuf, acc_e.at[pl.ds(ebase + k * _CH, _CH)])
        plsc.subcore_barrier()

        def s2(j, carry):
            pltpu.sync_copy(v2d.at[jbase + j], ivb)
            pltpu.sync_copy(e2d.at[jbase + j], ieb)
            pltpu.sync_copy(acc_e.at[ieb], rows)
            pltpu.sync_copy(rows, acc_v.at[ivb], add=True)
            return carry
        lax.fori_loop(0, _JTILE, s2, 0)
        plsc.subcore_barrier()

        @pl.when(s < _NSUB - 1)
        def _():
            pltpu.sync_copy(acc_v.at[pl.ds(s * 624, 624)],
                            out.at[bg].at[pl.ds(s * 624, 624)])

        @pl.when(s == _NSUB - 1)
        def _():
            pltpu.sync_copy(acc_v.at[pl.ds(15 * 624, 640)],
                            out.at[bg].at[pl.ds(15 * 624, 640)])
        plsc.subcore_barrier()


_sc_call = pl.kernel(
    _sc_body,
    out_type=(
        jax.ShapeDtypeStruct((_NB, _N, _F), jnp.float32),
        jax.ShapeDtypeStruct((_NCORE, _ME, _F), jnp.float32),
    ),
    mesh=plsc.VectorSubcoreMesh(core_axis_name="c", subcore_axis_name="s"),
    compiler_params=pltpu.CompilerParams(use_tc_tiling_on_sc=False),
    scratch_types=[
        pltpu.VMEM_SHARED((_ME, _F), jnp.float32),
        pltpu.VMEM_SHARED((_NV, _F), jnp.float32),
        pltpu.VMEM((_BATCH,), jnp.int32),
        pltpu.VMEM((_BATCH,), jnp.int32),
        pltpu.VMEM((_BATCH, _F), jnp.float32),
        pltpu.VMEM((_CH, _F), jnp.float32),
        pltpu.VMEM((_CH, _F), jnp.float32),
    ],
)


def kernel(X, vertex, edges, W, eps):
    Xh = _matmul(X, W)

    xh_pad = jnp.concatenate(
        [Xh, jnp.zeros((_NP - _N, _HID), jnp.float32)], axis=0)
    xh_b = xh_pad.reshape(_NP, _NB, _F).transpose(1, 0, 2)

    pad = _EPAD - _E
    v2d = jnp.concatenate(
        [vertex.astype(jnp.int32), jnp.full((pad,), _N, jnp.int32)]
    ).reshape(_EPAD // _BATCH, _BATCH)
    e2d = jnp.concatenate(
        [edges.astype(jnp.int32), jnp.full((pad,), _M, jnp.int32)]
    ).reshape(_EPAD // _BATCH, _BATCH)

    zsrc = jnp.zeros((_CH, _F), jnp.float32)
    osrc = jnp.ones((_BATCH, _F), jnp.float32)

    Xv_b, _unused_invc = _sc_call(xh_b, v2d, e2d, zsrc, osrc)
    Xv = Xv_b.transpose(1, 0, 2).reshape(_N, _HID)
    return _epilogue(eps, Xh, Xv)


# speedup vs baseline: 2.3815x; 1.0403x over previous
import functools
import jax
import jax.numpy as jnp
from jax import lax
from jax.experimental import pallas as pl
from jax.experimental.pallas import tpu as pltpu
from jax.experimental.pallas import tpu_sc as plsc

_N = 10000
_E = 320000
_M = 80000
_IN = 128
_HID = 256

_F = 16
_NB = _HID // _F
_NCORE = 2
_NSUB = 16
_BATCH = 512
_JTILE = 40
_EPAD = _JTILE * _NSUB * _BATCH
_ME = 81920
_CH = 512
_ECH = _ME // _NSUB // _CH
_NV = 10112
_NP = 10016


def _mm_kernel(x_ref, w_ref, o_ref):
    o_ref[...] = jnp.dot(x_ref[...], w_ref[...],
                         preferred_element_type=jnp.float32)


def _matmul(X, W):
    BM = 1000
    return pl.pallas_call(
        _mm_kernel,
        grid=(_N // BM,),
        in_specs=[
            pl.BlockSpec((BM, _IN), lambda i: (i, 0)),
            pl.BlockSpec((_IN, _HID), lambda i: (0, 0)),
        ],
        out_specs=pl.BlockSpec((BM, _HID), lambda i: (i, 0)),
        out_shape=jax.ShapeDtypeStruct((_N, _HID), jnp.float32),
    )(X, W)


def _ep_kernel(eps_ref, xh_ref, xv_ref, o_ref):
    o = (1.0 + eps_ref[0]) * xh_ref[...] + xv_ref[...]
    ss = jnp.sum(o * o, axis=1, keepdims=True)
    rn = jnp.sqrt(ss)
    scale = jnp.where(rn > 0, 1.0 / rn, 0.0)
    o_ref[...] = o * scale


def _epilogue(eps, Xh, Xv):
    BM = 1000
    return pl.pallas_call(
        _ep_kernel,
        grid=(_N // BM,),
        in_specs=[
            pl.BlockSpec(memory_space=pltpu.SMEM),
            pl.BlockSpec((BM, _HID), lambda i: (i, 0)),
            pl.BlockSpec((BM, _HID), lambda i: (i, 0)),
        ],
        out_specs=pl.BlockSpec((BM, _HID), lambda i: (i, 0)),
        out_shape=jax.ShapeDtypeStruct((_N, _HID), jnp.float32),
    )(eps, Xh, Xv)


def _sc_body(xh, v2d, e2d, zsrc, osrc, out, invc,
             acc_e, acc_v, ivb, ieb, rows, sbuf, jbuf, sem_g, sem_s):
    c = lax.axis_index("c")
    s = lax.axis_index("s")

    ebase = s * (_ME // _NSUB)
    jbase = s * _JTILE

    def zero_acc_e():
        pltpu.sync_copy(zsrc, sbuf)
        for k in range(_ECH):
            pltpu.sync_copy(sbuf, acc_e.at[pl.ds(ebase + k * _CH, _CH)])

    zero_acc_e()
    plsc.subcore_barrier()

    pltpu.sync_copy(osrc, rows.at[0])

    def cbody(j, carry):
        pltpu.sync_copy(e2d.at[jbase + j], ieb.at[0])
        pltpu.sync_copy(rows.at[0], acc_e.at[ieb.at[0]], add=True)
        return carry
    lax.fori_loop(0, _JTILE, cbody, 0)
    plsc.subcore_barrier()

    def run_pass(gtable, gidx_hbm, sidx_hbm, gidx, sidx, dacc):
        def load_idx(j, b):
            pltpu.sync_copy(gidx_hbm.at[jbase + j], gidx.at[b])
            pltpu.sync_copy(sidx_hbm.at[jbase + j], sidx.at[b])

        def issue_g(b):
            pltpu.async_copy(gtable.at[gidx.at[b]], rows.at[b], sem_g)

        def wait_g(b):
            pltpu.make_async_copy(gtable.at[gidx.at[b]], rows.at[b],
                                  sem_g).wait()

        def issue_s(b):
            pltpu.async_copy(rows.at[b], dacc.at[sidx.at[b]], sem_s, add=True)

        def wait_s(b):
            pltpu.make_async_copy(rows.at[b], dacc.at[sidx.at[b]],
                                  sem_s).wait()

        load_idx(0, 0)
        issue_g(0)
        load_idx(1, 1)
        issue_g(1)
        wait_g(0)
        issue_s(0)

        def pbody(jj, carry):
            for b in (0, 1):
                j = jj * 2 + b
                wait_s(b)
                load_idx(j, b)
                issue_g(b)
                wait_g(1 - b)
                issue_s(1 - b)
            return carry
        lax.fori_loop(1, _JTILE // 2, pbody, 0)

        wait_g(1)
        issue_s(1)
        wait_s(0)
        wait_s(1)

    for k in range(_ECH):
        pltpu.sync_copy(acc_e.at[pl.ds(ebase + k * _CH, _CH)], sbuf)

        def gbody(r, carry):
            sbuf[r, :] = 1.0 / jnp.maximum(sbuf[r, :], 1.0)
            return carry
        lax.fori_loop(0, _CH, gbody, 0)
        pltpu.sync_copy(sbuf, invc.at[c].at[pl.ds(ebase + k * _CH, _CH)])
    plsc.subcore_barrier()

    for bl in range(_NB // _NCORE):
        bg = c * (_NB // _NCORE) + bl
        zero_acc_e()
        pltpu.sync_copy(sbuf, acc_v.at[pl.ds(s * 632, _CH)])
        pltpu.sync_copy(sbuf.at[pl.ds(0, 120)],
                        acc_v.at[pl.ds(s * 632 + _CH, 120)])
        plsc.subcore_barrier()

        run_pass(xh.at[bg], v2d, e2d, ivb, ieb, acc_e)
        plsc.subcore_barrier()

        for k in range(_ECH):
            pltpu.sync_copy(acc_e.at[pl.ds(ebase + k * _CH, _CH)], sbuf)
            pltpu.sync_copy(invc.at[c].at[pl.ds(ebase + k * _CH, _CH)], jbuf)

            def scbody(r, carry):
                sbuf[r, :] = sbuf[r, :] * jbuf[r, :]
                return carry
            lax.fori_loop(0, _CH, scbody, 0)
            pltpu.sync_copy(sbuf, acc_e.at[pl.ds(ebase + k * _CH, _CH)])
        plsc.subcore_barrier()

        run_pass(acc_e, e2d, v2d, ieb, ivb, acc_v)
        plsc.subcore_barrier()

        @pl.when(s < _NSUB - 1)
        def _():
            pltpu.sync_copy(acc_v.at[pl.ds(s * 624, 624)],
                            out.at[bg].at[pl.ds(s * 624, 624)])

        @pl.when(s == _NSUB - 1)
        def _():
            pltpu.sync_copy(acc_v.at[pl.ds(15 * 624, 640)],
                            out.at[bg].at[pl.ds(15 * 624, 640)])
        plsc.subcore_barrier()


_sc_call = pl.kernel(
    _sc_body,
    out_type=(
        jax.ShapeDtypeStruct((_NB, _N, _F), jnp.float32),
        jax.ShapeDtypeStruct((_NCORE, _ME, _F), jnp.float32),
    ),
    mesh=plsc.VectorSubcoreMesh(core_axis_name="c", subcore_axis_name="s"),
    compiler_params=pltpu.CompilerParams(use_tc_tiling_on_sc=False),
    scratch_types=[
        pltpu.VMEM_SHARED((_ME, _F), jnp.float32),
        pltpu.VMEM_SHARED((_NV, _F), jnp.float32),
        pltpu.VMEM((2, _BATCH), jnp.int32),
        pltpu.VMEM((2, _BATCH), jnp.int32),
        pltpu.VMEM((2, _BATCH, _F), jnp.float32),
        pltpu.VMEM((_CH, _F), jnp.float32),
        pltpu.VMEM((_CH, _F), jnp.float32),
        pltpu.SemaphoreType.DMA,
        pltpu.SemaphoreType.DMA,
    ],
)


def kernel(X, vertex, edges, W, eps):
    Xh = _matmul(X, W)

    xh_pad = jnp.concatenate(
        [Xh, jnp.zeros((_NP - _N, _HID), jnp.float32)], axis=0)
    xh_b = xh_pad.reshape(_NP, _NB, _F).transpose(1, 0, 2)

    pad = _EPAD - _E
    v2d = jnp.concatenate(
        [vertex.astype(jnp.int32), jnp.full((pad,), _N, jnp.int32)]
    ).reshape(_EPAD // _BATCH, _BATCH)
    e2d = jnp.concatenate(
        [edges.astype(jnp.int32), jnp.full((pad,), _M, jnp.int32)]
    ).reshape(_EPAD // _BATCH, _BATCH)

    zsrc = jnp.zeros((_CH, _F), jnp.float32)
    osrc = jnp.ones((_BATCH, _F), jnp.float32)

    Xv_b, _unused_invc = _sc_call(xh_b, v2d, e2d, zsrc, osrc)
    Xv = Xv_b.transpose(1, 0, 2).reshape(_N, _HID)
    return _epilogue(eps, Xh, Xv)
